# Initial kernel scaffold; baseline (speedup 1.0000x reference)
import jax
import jax.numpy as jnp
from jax.experimental import pallas as pl


def kernel(x, edge_index, edge_type, W1, root1, b1, gamma1, beta1, W2, root2, b2):
    raise NotImplementedError("write your pallas kernel here")


# speedup vs baseline: 11.2673x; 11.2673x over previous
import functools

import jax
import jax.numpy as jnp
from jax import lax
from jax.experimental import pallas as pl
from jax.experimental.pallas import tpu as pltpu
from jax.experimental.pallas import tpu_sc as plsc

KCH = 128
LN_EPS = 1e-5


def _transform_body(x_ref, w_ref, y_ref):
    y_ref[...] = jnp.dot(x_ref[...], w_ref[0],
                         preferred_element_type=jnp.float32)


def _transform(x, w_ext, bn):
    n, c = x.shape
    k = w_ext.shape[0]
    nb = n // bn
    return pl.pallas_call(
        _transform_body,
        grid=(nb, k),
        in_specs=[
            pl.BlockSpec((bn, c), lambda i, r: (i, 0)),
            pl.BlockSpec((1, c, c), lambda i, r: (r, 0, 0)),
        ],
        out_specs=pl.BlockSpec((bn, c), lambda i, r, _nb=nb: (r * _nb + i, 0)),
        out_shape=jax.ShapeDtypeStruct((k * n, c), jnp.float32),
    )(x, w_ext)


def _mid_body(acc_ref, r1_ref, b_ref, g_ref, be_ref, w_ref, y_ref):
    s = acc_ref[0] + acc_ref[1] + r1_ref[...] + b_ref[...]
    mu = jnp.mean(s, axis=-1, keepdims=True)
    var = jnp.mean((s - mu) ** 2, axis=-1, keepdims=True)
    h = (s - mu) / jnp.sqrt(var + LN_EPS) * g_ref[...] + be_ref[...]
    h = jnp.maximum(h, 0.0)
    y_ref[...] = jnp.dot(h, w_ref[0], preferred_element_type=jnp.float32)


def _mid(acc, y1, b1, g1, be1, w2_ext, n, bn):
    c = y1.shape[1]
    k = w2_ext.shape[0]
    nb = n // bn
    return pl.pallas_call(
        _mid_body,
        grid=(nb, k),
        in_specs=[
            pl.BlockSpec((2, bn, c), lambda i, r: (0, i, 0)),
            pl.BlockSpec((bn, c), lambda i, r, _nb=nb, _k=k: ((_k - 1) * _nb + i, 0)),
            pl.BlockSpec((c,), lambda i, r: (0,)),
            pl.BlockSpec((c,), lambda i, r: (0,)),
            pl.BlockSpec((c,), lambda i, r: (0,)),
            pl.BlockSpec((1, c, c), lambda i, r: (r, 0, 0)),
        ],
        out_specs=pl.BlockSpec((bn, c), lambda i, r, _nb=nb: (r * _nb + i, 0)),
        out_shape=jax.ShapeDtypeStruct((k * n, c), jnp.float32),
    )(acc, y1, b1, g1, be1, w2_ext)


def _final_body(acc_ref, r2_ref, b_ref, out_ref):
    out_ref[...] = acc_ref[0] + acc_ref[1] + r2_ref[...] + b_ref[...]


def _final(acc, y2, b2, n, bn):
    c = y2.shape[1]
    k = y2.shape[0] // n
    nb = n // bn
    return pl.pallas_call(
        _final_body,
        grid=(nb,),
        in_specs=[
            pl.BlockSpec((2, bn, c), lambda i: (0, i, 0)),
            pl.BlockSpec((bn, c), lambda i, _nb=nb, _k=k: ((_k - 1) * _nb + i, 0)),
            pl.BlockSpec((c,), lambda i: (0,)),
        ],
        out_specs=pl.BlockSpec((bn, c), lambda i: (i, 0)),
        out_shape=jax.ShapeDtypeStruct((n, c), jnp.float32),
    )(acc, y2, b2)


def _zero_vec(ref, nwords):
    def st(j, _):
        ref[pl.ds(j * 16, 16)] = jnp.zeros((16,), jnp.float32)
        return 0
    lax.fori_loop(0, nwords // 16, st, 0)


def _fill_ones(ref, nwords):
    def st(j, _):
        ref[pl.ds(j * 16, 16)] = jnp.ones((16,), jnp.float32)
        return 0
    lax.fori_loop(0, nwords // 16, st, 0)


def _agg_value_pass(src_hbm, dst_hbm, et_hbm, y_hbm, acc_sh, inv_sh,
                    rows_v, sbuf, dbuf, tbuf, ibuf, widx, wbuf, sem,
                    wid, nw, n, nchunks, c):
    nk = nchunks // nw
    cl = c // 16

    def chunk(k_, _):
        base = (wid + k_ * nw) * KCH
        pltpu.sync_copy(src_hbm.at[pl.ds(base, KCH)], sbuf)
        pltpu.sync_copy(dst_hbm.at[pl.ds(base, KCH)], dbuf)
        pltpu.sync_copy(et_hbm.at[pl.ds(base, KCH)], tbuf)

        def gw(j, _):
            sl = pl.ds(j * 16, 16)
            t = tbuf[sl]
            ibuf[sl] = t * n + sbuf[sl]
            widx[sl] = t * n + dbuf[sl]
            return 0
        lax.fori_loop(0, KCH // 16, gw, 0)

        pltpu.async_copy(y_hbm.at[ibuf], rows_v, sem).wait()
        pltpu.async_copy(inv_sh.at[widx], wbuf, sem).wait()

        def scale(g, _):
            w16 = wbuf[pl.ds(g * 16, 16)]
            for e_ in range(16):
                i = g * 16 + e_
                w = w16[e_]
                for j in range(cl):
                    sl = pl.ds(j * 16, 16)
                    rows_v[i, sl] = rows_v[i, sl] * w
            return 0
        lax.fori_loop(0, KCH // 16, scale, 0)

        pltpu.sync_copy(rows_v, acc_sh.at[dbuf], add=True)
        return 0

    lax.fori_loop(0, nk, chunk, 0)


def _acc_writeout(acc_sh, rows_v, acc_hbm, core, sid, asl, c):
    for z in range(asl // KCH):
        row = sid * asl + z * KCH
        pltpu.sync_copy(acc_sh.at[pl.ds(row, KCH)], rows_v)
        pltpu.sync_copy(rows_v, acc_hbm.at[core, pl.ds(row, KCH)])


def _sc_first_body(src_hbm, dst_hbm, et_hbm, y_hbm, acc_hbm, inv_hbm,
                   cnt_sh, acc_sh, rows_v, sbuf, dbuf, tbuf, ibuf, widx,
                   wbuf, ones_v, zbuf, sem, *, n, np_, rnp, nchunks, c,
                   nc, ns):
    core = lax.axis_index("c")
    sid = lax.axis_index("s")
    wid = core * ns + sid
    nw = nc * ns
    csl = rnp // ns
    asl = np_ // ns

    _zero_vec(zbuf, csl)
    _fill_ones(ones_v, KCH)

    def zr(i, _):
        for j in range(c // 16):
            rows_v[i, pl.ds(j * 16, 16)] = jnp.zeros((16,), jnp.float32)
        return 0
    lax.fori_loop(0, KCH, zr, 0)
    pltpu.sync_copy(zbuf, cnt_sh.at[pl.ds(sid * csl, csl)])
    for z in range(asl // KCH):
        pltpu.sync_copy(rows_v, acc_sh.at[pl.ds(sid * asl + z * KCH, KCH)])
    plsc.subcore_barrier()

    nk = nchunks // ns

    def count_chunk(k_, _):
        base = (sid + k_ * ns) * KCH
        pltpu.sync_copy(dst_hbm.at[pl.ds(base, KCH)], dbuf)
        pltpu.sync_copy(et_hbm.at[pl.ds(base, KCH)], tbuf)

        def cidx(j, _):
            sl = pl.ds(j * 16, 16)
            ibuf[sl] = tbuf[sl] * n + dbuf[sl]
            return 0
        lax.fori_loop(0, KCH // 16, cidx, 0)
        pltpu.sync_copy(ones_v, cnt_sh.at[ibuf], add=True)
        return 0

    lax.fori_loop(0, nk, count_chunk, 0)
    plsc.subcore_barrier()

    tsl = pl.ds(sid * csl, csl)
    pltpu.sync_copy(cnt_sh.at[tsl], zbuf)

    def invb(j, _):
        sl = pl.ds(j * 16, 16)
        zbuf[sl] = 1.0 / jnp.maximum(zbuf[sl], 1.0)
        return 0
    lax.fori_loop(0, csl // 16, invb, 0)
    pltpu.sync_copy(zbuf, cnt_sh.at[tsl])

    @pl.when(core == 0)
    def _():
        pltpu.sync_copy(zbuf, inv_hbm.at[tsl])

    plsc.subcore_barrier()

    _agg_value_pass(src_hbm, dst_hbm, et_hbm, y_hbm, acc_sh, cnt_sh,
                    rows_v, sbuf, dbuf, tbuf, ibuf, widx, wbuf, sem,
                    wid, nw, n, nchunks, c)
    plsc.subcore_barrier()

    _acc_writeout(acc_sh, rows_v, acc_hbm, core, sid, asl, c)


def _sc_second_body(src_hbm, dst_hbm, et_hbm, y_hbm, inv_hbm, acc_hbm,
                    inv_sh, acc_sh, rows_v, sbuf, dbuf, tbuf, ibuf, widx,
                    wbuf, cbuf, sem, *, n, np_, rnp, nchunks, c, nc, ns):
    core = lax.axis_index("c")
    sid = lax.axis_index("s")
    wid = core * ns + sid
    nw = nc * ns
    asl = np_ // ns
    csl = rnp // ns

    def zr(i, _):
        for j in range(c // 16):
            rows_v[i, pl.ds(j * 16, 16)] = jnp.zeros((16,), jnp.float32)
        return 0
    lax.fori_loop(0, KCH, zr, 0)
    for z in range(asl // KCH):
        pltpu.sync_copy(rows_v, acc_sh.at[pl.ds(sid * asl + z * KCH, KCH)])
    tsl = pl.ds(sid * csl, csl)
    pltpu.sync_copy(inv_hbm.at[tsl], cbuf)
    pltpu.sync_copy(cbuf, inv_sh.at[tsl])
    plsc.subcore_barrier()

    _agg_value_pass(src_hbm, dst_hbm, et_hbm, y_hbm, acc_sh, inv_sh,
                    rows_v, sbuf, dbuf, tbuf, ibuf, widx, wbuf, sem,
                    wid, nw, n, nchunks, c)
    plsc.subcore_barrier()

    _acc_writeout(acc_sh, rows_v, acc_hbm, core, sid, asl, c)


def kernel(x, edge_index, edge_type, W1, root1, b1, gamma1, beta1,
           W2, root2, b2):
    n, c = x.shape
    e = edge_type.shape[0]
    r = W1.shape[0]

    info = plsc.get_sparse_core_info()
    nc, ns = info.num_cores, info.num_subcores
    nw = nc * ns

    asl = ((-(-n // ns)) + KCH - 1) // KCH * KCH
    np_ = asl * ns
    csl = (-(-(r * n) // ns) + 15) // 16 * 16
    rnp = csl * ns
    nchunks = -(-e // (nw * KCH)) * nw
    e_pad = nchunks * KCH

    src = edge_index[0].astype(jnp.int32)
    dst = edge_index[1].astype(jnp.int32)
    et = edge_type.astype(jnp.int32)
    if e_pad > e:
        pad = e_pad - e
        src = jnp.concatenate([src, jnp.zeros((pad,), jnp.int32)])
        dst = jnp.concatenate([dst, jnp.full((pad,), n, jnp.int32)])
        et = jnp.concatenate([et, jnp.full((pad,), r - 1, jnp.int32)])

    w1_ext = jnp.concatenate([W1, root1[None]], axis=0)
    w2_ext = jnp.concatenate([W2, root2[None]], axis=0)

    bn = 400 if n % 400 == 0 else 100
    y1 = _transform(x, w1_ext, bn)

    mesh = plsc.VectorSubcoreMesh(core_axis_name="c", subcore_axis_name="s")
    f32 = jnp.float32
    i32 = jnp.int32

    common_scratch = [
        pltpu.VMEM((KCH, c), f32),
        pltpu.VMEM((KCH,), i32),
        pltpu.VMEM((KCH,), i32),
        pltpu.VMEM((KCH,), i32),
        pltpu.VMEM((KCH,), i32),
        pltpu.VMEM((KCH,), i32),
        pltpu.VMEM((KCH,), f32),
    ]

    first = pl.kernel(
        functools.partial(_sc_first_body, n=n, np_=np_, rnp=rnp,
                          nchunks=nchunks, c=c, nc=nc, ns=ns),
        out_type=[jax.ShapeDtypeStruct((2, np_, c), f32),
                  jax.ShapeDtypeStruct((rnp,), f32)],
        mesh=mesh,
        scratch_types=[
            pltpu.VMEM_SHARED((rnp,), f32),
            pltpu.VMEM_SHARED((np_, c), f32),
            *common_scratch,
            pltpu.VMEM((KCH,), f32),
            pltpu.VMEM((rnp // ns,), f32),
            pltpu.SemaphoreType.DMA,
        ],
    )
    acc1, inv = first(src, dst, et, y1)

    y2 = _mid(acc1, y1, b1, gamma1, beta1, w2_ext, n, bn)

    second = pl.kernel(
        functools.partial(_sc_second_body, n=n, np_=np_, rnp=rnp,
                          nchunks=nchunks, c=c, nc=nc, ns=ns),
        out_type=jax.ShapeDtypeStruct((2, np_, c), f32),
        mesh=mesh,
        scratch_types=[
            pltpu.VMEM_SHARED((rnp,), f32),
            pltpu.VMEM_SHARED((np_, c), f32),
            *common_scratch,
            pltpu.VMEM((rnp // ns,), f32),
            pltpu.SemaphoreType.DMA,
        ],
    )
    acc2 = second(src, dst, et, y2, inv)

    return _final(acc2, y2, b2, n, bn)
